# Initial kernel scaffold; baseline (speedup 1.0000x reference)
import functools

import jax
import jax.numpy as jnp
from jax import lax
from jax.experimental import pallas as pl
from jax.experimental.pallas import tpu as pltpu
from jax.experimental.pallas import tpu_sc as plsc

NC = 2
NS = 16
NW = NC * NS
K = 80
T = 16


def _sc_body(n, e, d, meta_hbm, x_hbm, acc_out, cnt_out,
             metav, rowsv, onehot, accsh, cntsh, sem):
    cid = lax.axis_index("c")
    sid = lax.axis_index("s")
    wid = sid * NC + cid

    rows_per_tile = n // NS
    epw = e // NW
    chunks = epw // K

    zero16 = jnp.zeros((16,), jnp.float32)
    ones16 = jnp.ones((16,), jnp.float32)
    iota16 = lax.iota(jnp.int32, 16)

    @pl.loop(0, K * (d // 16))
    def _zrows(i):
        rowsv[i // (d // 16), pl.ds((i % (d // 16)) * 16, 16)] = zero16

    @pl.loop(0, K)
    def _zoh(i):
        onehot[i, :] = zero16

    base_r = sid * rows_per_tile
    nfull = rows_per_tile // K
    rem = rows_per_tile - nfull * K
    for c in range(nfull):
        pltpu.sync_copy(rowsv, accsh.at[pl.ds(base_r + c * K, K)])
        pltpu.sync_copy(onehot, cntsh.at[pl.ds(base_r + c * K, K)])
    if rem:
        pltpu.sync_copy(rowsv.at[pl.ds(0, rem)],
                        accsh.at[pl.ds(base_r + nfull * K, rem)])
        pltpu.sync_copy(onehot.at[pl.ds(0, rem)],
                        cntsh.at[pl.ds(base_r + nfull * K, rem)])
    plsc.subcore_barrier()

    @pl.loop(0, chunks)
    def _edges(j):
        start = wid * epw + j * K
        pltpu.sync_copy(meta_hbm.at[:, pl.ds(start, K)], metav)
        pltpu.async_copy(x_hbm.at[metav.at[1]], rowsv, sem).wait()
        for g in range(K // 16):
            t = metav[2, pl.ds(g * 16, 16)] * 3 + metav[3, pl.ds(g * 16, 16)]
            plsc.store_scatter(onehot, [iota16 + g * 16, t], ones16)
        pltpu.sync_copy(rowsv, accsh.at[metav.at[0]], add=True)
        pltpu.sync_copy(onehot, cntsh.at[metav.at[0]], add=True)
        for g in range(K // 16):
            t = metav[2, pl.ds(g * 16, 16)] * 3 + metav[3, pl.ds(g * 16, 16)]
            plsc.store_scatter(onehot, [iota16 + g * 16, t], zero16)

    plsc.subcore_barrier()

    pltpu.sync_copy(accsh.at[pl.ds(base_r, rows_per_tile)],
                    acc_out.at[cid, pl.ds(base_r, rows_per_tile)])
    pltpu.sync_copy(cntsh.at[pl.ds(base_r, rows_per_tile)],
                    cnt_out.at[cid, pl.ds(base_r, rows_per_tile)])


def _mlp_body(acc_ref, cnt_ref, x_ref, emb1_ref, emb2_ref,
              w1_ref, b1_ref, w2_ref, b2_ref, out_ref):
    tt = lax.broadcasted_iota(jnp.int32, (T, 5), 0)
    s1 = (tt // 3 == lax.broadcasted_iota(jnp.int32, (T, 5), 1)).astype(jnp.float32)
    td = lax.broadcasted_iota(jnp.int32, (T, 3), 0)
    s2 = (td % 3 == lax.broadcasted_iota(jnp.int32, (T, 3), 1)).astype(jnp.float32)
    embc = (jnp.dot(s1, emb1_ref[...], preferred_element_type=jnp.float32)
            + jnp.dot(s2, emb2_ref[...], preferred_element_type=jnp.float32))
    c0 = emb1_ref[4:5, :] + emb2_ref[0:1, :]

    a = acc_ref[0] + acc_ref[1] + x_ref[...] + c0
    c = cnt_ref[0] + cnt_ref[1]
    m = a + jnp.dot(c, embc, preferred_element_type=jnp.float32)
    h = jnp.maximum(jnp.dot(m, w1_ref[...], preferred_element_type=jnp.float32)
                    + b1_ref[...], 0.0)
    out_ref[...] = (jnp.dot(h, w2_ref[...], preferred_element_type=jnp.float32)
                    + b2_ref[...])


@jax.jit
def kernel(x, edge_index, edge_attr, emb1, emb2, W1, b1, W2, b2):
    n, d = x.shape
    e = edge_index.shape[1]
    assert e % (NW * K) == 0 and n % NS == 0 and d % 16 == 0

    meta = jnp.concatenate(
        [edge_index.astype(jnp.int32), edge_attr.T.astype(jnp.int32)], axis=0)

    mesh = plsc.VectorSubcoreMesh(core_axis_name="c", subcore_axis_name="s")
    acc, cnt = pl.kernel(
        functools.partial(_sc_body, n, e, d),
        out_type=[jax.ShapeDtypeStruct((NC, n, d), jnp.float32),
                  jax.ShapeDtypeStruct((NC, n, T), jnp.float32)],
        mesh=mesh,
        scratch_types=[
            pltpu.VMEM((4, K), jnp.int32),
            pltpu.VMEM((K, d), jnp.float32),
            pltpu.VMEM((K, T), jnp.float32),
            pltpu.VMEM_SHARED((n, d), jnp.float32),
            pltpu.VMEM_SHARED((n, T), jnp.float32),
            pltpu.SemaphoreType.DMA,
        ],
    )(meta, x)

    rblk = 2000
    grid = n // rblk
    out = pl.pallas_call(
        _mlp_body,
        grid=(grid,),
        in_specs=[
            pl.BlockSpec((NC, rblk, d), lambda i: (0, i, 0)),
            pl.BlockSpec((NC, rblk, T), lambda i: (0, i, 0)),
            pl.BlockSpec((rblk, d), lambda i: (i, 0)),
            pl.BlockSpec(emb1.shape, lambda i: (0, 0)),
            pl.BlockSpec(emb2.shape, lambda i: (0, 0)),
            pl.BlockSpec(W1.shape, lambda i: (0, 0)),
            pl.BlockSpec((1, W1.shape[1]), lambda i: (0, 0)),
            pl.BlockSpec(W2.shape, lambda i: (0, 0)),
            pl.BlockSpec((1, W2.shape[1]), lambda i: (0, 0)),
        ],
        out_specs=pl.BlockSpec((rblk, d), lambda i: (i, 0)),
        out_shape=jax.ShapeDtypeStruct((n, d), jnp.float32),
    )(acc, cnt, x, emb1, emb2, W1, b1.reshape(1, -1), W2, b2.reshape(1, -1))
    return out


# speedup vs baseline: 2.9218x; 2.9218x over previous
import functools

import jax
import jax.numpy as jnp
from jax import lax
from jax.experimental import pallas as pl
from jax.experimental.pallas import tpu as pltpu
from jax.experimental.pallas import tpu_sc as plsc

NC = 2
NS = 16
NW = NC * NS
K = 128
T = 16


def _sc_body(n, e, d, meta_hbm, x_hbm, embc_hbm, acc_out,
             metav, rowsv, ebuf, tbuf, accsh, sem, sem2):
    cid = lax.axis_index("c")
    sid = lax.axis_index("s")
    wid = sid * NC + cid

    rows8 = (n // NS) // 8 * 8
    tail = n - rows8 * NS
    nchunks = e // K
    my_chunks = (nchunks - 1 - wid) // NW + 1

    zero16 = jnp.zeros((16,), jnp.float32)

    @pl.loop(0, K * (d // 16))
    def _zrows(i):
        rowsv[i // (d // 16), pl.ds((i % (d // 16)) * 16, 16)] = zero16

    base_r = sid * rows8
    nfull = rows8 // K
    rem = rows8 - nfull * K
    for c in range(nfull):
        pltpu.sync_copy(rowsv, accsh.at[pl.ds(base_r + c * K, K)])
    if rem:
        pltpu.sync_copy(rowsv.at[pl.ds(0, rem)],
                        accsh.at[pl.ds(base_r + nfull * K, rem)])
    if tail:
        @pl.when(sid == NS - 1)
        def _ztail():
            pltpu.sync_copy(rowsv.at[pl.ds(0, tail)],
                            accsh.at[pl.ds(rows8 * NS, tail)])
    plsc.subcore_barrier()

    @pl.loop(0, my_chunks)
    def _edges(j):
        start = (j * NW + wid) * K
        pltpu.sync_copy(meta_hbm.at[:, pl.ds(start, K)], metav)
        cp1 = pltpu.async_copy(x_hbm.at[metav.at[1]], rowsv, sem)

        @pl.loop(0, K // 16)
        def _t(g):
            tbuf[pl.ds(g * 16, 16)] = (metav[2, pl.ds(g * 16, 16)] * 3
                                       + metav[3, pl.ds(g * 16, 16)])
        cp2 = pltpu.async_copy(embc_hbm.at[tbuf], ebuf, sem2)

        cp1.wait()
        pltpu.sync_copy(rowsv, accsh.at[metav.at[0]], add=True)
        cp2.wait()
        pltpu.sync_copy(ebuf, accsh.at[metav.at[0]], add=True)

    plsc.subcore_barrier()

    pltpu.sync_copy(accsh.at[pl.ds(base_r, rows8)],
                    acc_out.at[cid, pl.ds(base_r, rows8)])
    if tail:
        @pl.when(sid == NS - 1)
        def _ctail():
            pltpu.sync_copy(accsh.at[pl.ds(rows8 * NS, tail)],
                            acc_out.at[cid, pl.ds(rows8 * NS, tail)])


def _mlp_body(acc_ref, x_ref, emb1_ref, emb2_ref,
              w1_ref, b1_ref, w2_ref, b2_ref, out_ref):
    c0 = emb1_ref[4:5, :] + emb2_ref[0:1, :]
    m = acc_ref[0] + acc_ref[1] + x_ref[...] + c0
    h = jnp.maximum(jnp.dot(m, w1_ref[...], preferred_element_type=jnp.float32)
                    + b1_ref[...], 0.0)
    out_ref[...] = (jnp.dot(h, w2_ref[...], preferred_element_type=jnp.float32)
                    + b2_ref[...])


@jax.jit
def kernel(x, edge_index, edge_attr, emb1, emb2, W1, b1, W2, b2):
    n, d = x.shape
    e = edge_index.shape[1]
    assert e % K == 0 and n % NS == 0 and d % 16 == 0

    meta = jnp.concatenate(
        [edge_index.astype(jnp.int32), edge_attr.T.astype(jnp.int32)], axis=0)
    ti = jnp.arange(T)
    embc = jnp.where((ti < 15)[:, None],
                     emb1[jnp.minimum(ti // 3, 4)] + emb2[ti % 3], 0.0)

    mesh = plsc.VectorSubcoreMesh(core_axis_name="c", subcore_axis_name="s")
    acc = pl.kernel(
        functools.partial(_sc_body, n, e, d),
        out_type=jax.ShapeDtypeStruct((NC, n, d), jnp.float32),
        mesh=mesh,
        scratch_types=[
            pltpu.VMEM((4, K), jnp.int32),
            pltpu.VMEM((K, d), jnp.float32),
            pltpu.VMEM((K, d), jnp.float32),
            pltpu.VMEM((K,), jnp.int32),
            pltpu.VMEM_SHARED((n, d), jnp.float32),
            pltpu.SemaphoreType.DMA,
            pltpu.SemaphoreType.DMA,
        ],
    )(meta, x, embc)

    rblk = 2000
    grid = n // rblk
    out = pl.pallas_call(
        _mlp_body,
        grid=(grid,),
        in_specs=[
            pl.BlockSpec((NC, rblk, d), lambda i: (0, i, 0)),
            pl.BlockSpec((rblk, d), lambda i: (i, 0)),
            pl.BlockSpec(emb1.shape, lambda i: (0, 0)),
            pl.BlockSpec(emb2.shape, lambda i: (0, 0)),
            pl.BlockSpec(W1.shape, lambda i: (0, 0)),
            pl.BlockSpec((1, W1.shape[1]), lambda i: (0, 0)),
            pl.BlockSpec(W2.shape, lambda i: (0, 0)),
            pl.BlockSpec((1, W2.shape[1]), lambda i: (0, 0)),
        ],
        out_specs=pl.BlockSpec((rblk, d), lambda i: (i, 0)),
        out_shape=jax.ShapeDtypeStruct((n, d), jnp.float32),
    )(acc, x, emb1, emb2, W1, b1.reshape(1, -1), W2, b2.reshape(1, -1))
    return out
